# baseline (speedup 1.0000x reference)
import functools

import jax
import jax.numpy as jnp
from jax import lax
from jax.experimental import pallas as pl
from jax.experimental.pallas import tpu as pltpu
from jax.experimental.pallas import tpu_sc as plsc

B, A, C = 16, 64, 4096
L = 16
ROWS, COLS = 8, 128
CHUNKS = COLS // L


@functools.partial(
    pl.kernel,
    mesh=plsc.VectorSubcoreMesh(core_axis_name="c", subcore_axis_name="s"),
    out_type=jax.ShapeDtypeStruct((B,), jnp.float32),
    scratch_types=[
        pltpu.VMEM((A * B,), jnp.int32),
        pltpu.VMEM((ROWS, COLS), jnp.int32),
        pltpu.VMEM((ROWS, COLS), jnp.float32),
        pltpu.VMEM((B,), jnp.float32),
        pltpu.SemaphoreType.DMA,
    ],
)
def _arg_loss_body(yt_hbm, yp_hbm, out_hbm, yt_v, idx_v, val_v, out_v, sem):
    cid = lax.axis_index("c")
    sid = lax.axis_index("s")

    @pl.when((cid == 0) & (sid == 0))
    def _():
        pltpu.sync_copy(yt_hbm, yt_v)
        lane = lax.iota(jnp.int32, L)
        for a in range(A):
            yt16 = yt_v[pl.ds(a * L, L)]
            safe = jnp.maximum(yt16, 0)
            flat = (lane * A + a) * C + safe
            idx_v[a // CHUNKS, pl.ds((a % CHUNKS) * L, L)] = flat
        copies = [
            pltpu.async_copy(yp_hbm.at[idx_v.at[j]], val_v.at[j], sem)
            for j in range(ROWS)
        ]
        for cp in copies:
            cp.wait()
        acc = jnp.zeros((L,), jnp.float32)
        for a in range(A):
            g = val_v[a // CHUNKS, pl.ds((a % CHUNKS) * L, L)]
            m = yt_v[pl.ds(a * L, L)] >= 0
            acc = acc + jnp.where(m, g, jnp.zeros((L,), jnp.float32))
        out_v[...] = -acc
        pltpu.sync_copy(out_v, out_hbm)


def kernel(y_pred, y_true):
    yt = jnp.squeeze(y_true, axis=1).astype(jnp.int32).T.reshape(-1)
    yp = y_pred.reshape(-1)
    return _arg_loss_body(yt, yp)


# speedup vs baseline: 1.7214x; 1.7214x over previous
import functools

import jax
import jax.numpy as jnp
from jax import lax
from jax.experimental import pallas as pl
from jax.experimental.pallas import tpu as pltpu
from jax.experimental.pallas import tpu_sc as plsc

B, A, C = 16, 64, 4096
L = 16
ROWS, COLS = 8, 128
CHUNKS = COLS // L


@functools.partial(
    pl.kernel,
    mesh=plsc.VectorSubcoreMesh(core_axis_name="c", subcore_axis_name="s"),
    out_type=jax.ShapeDtypeStruct((B,), jnp.float32),
    scratch_types=[
        pltpu.VMEM((A * B,), jnp.int32),
        pltpu.VMEM((ROWS, COLS), jnp.int32),
        pltpu.VMEM((ROWS, COLS), jnp.float32),
        pltpu.VMEM((B,), jnp.float32),
        pltpu.SemaphoreType.DMA,
    ],
)
def _arg_loss_body(yt_hbm, yp_hbm, out_hbm, yt_v, idx_v, val_v, out_v, sem):
    cid = lax.axis_index("c")
    sid = lax.axis_index("s")

    @pl.when((cid == 0) & (sid == 0))
    def _():
        pltpu.sync_copy(yt_hbm, yt_v)
        lane = lax.iota(jnp.int32, L)
        lane_base = lane * (A * C)
        for a in range(A):
            yt16 = yt_v[pl.ds(a * L, L)]
            safe = jnp.maximum(yt16, 0)
            a_base = (a // 8) * 32768 + (a % 8) * 128
            flat = (
                lane_base + a_base
                + lax.shift_left(lax.shift_right_logical(safe, 7), 10)
                + lax.bitwise_and(safe, 127)
            )
            idx_v[a // CHUNKS, pl.ds((a % CHUNKS) * L, L)] = flat
        copies = [
            pltpu.async_copy(yp_hbm.at[idx_v.at[j]], val_v.at[j], sem)
            for j in range(ROWS)
        ]
        for cp in copies:
            cp.wait()
        acc = jnp.zeros((L,), jnp.float32)
        for a in range(A):
            g = val_v[a // CHUNKS, pl.ds((a % CHUNKS) * L, L)]
            m = yt_v[pl.ds(a * L, L)] >= 0
            acc = acc + jnp.where(m, g, jnp.zeros((L,), jnp.float32))
        out_v[...] = -acc
        pltpu.sync_copy(out_v, out_hbm)


def kernel(y_pred, y_true):
    yt = jnp.squeeze(y_true, axis=1).astype(jnp.int32).T.reshape(-1)
    yp = y_pred.reshape(B * A // 8, 8, C // 128, 128).transpose(0, 2, 1, 3).reshape(-1)
    return _arg_loss_body(yt, yp)


# speedup vs baseline: 2.0152x; 1.1707x over previous
import functools

import jax
import jax.numpy as jnp
from jax import lax
from jax.experimental import pallas as pl
from jax.experimental.pallas import tpu as pltpu
from jax.experimental.pallas import tpu_sc as plsc

B, A, C = 16, 64, 4096
L = 16
NS = 16
PER = A // NS


@functools.partial(
    pl.kernel,
    mesh=plsc.VectorSubcoreMesh(
        core_axis_name="c", subcore_axis_name="s", num_cores=1
    ),
    out_type=jax.ShapeDtypeStruct((B,), jnp.float32),
    scratch_types=[
        pltpu.VMEM((PER * L,), jnp.int32),
        pltpu.VMEM((PER * L,), jnp.int32),
        pltpu.VMEM((PER * L,), jnp.float32),
        pltpu.VMEM((L,), jnp.float32),
        pltpu.VMEM((NS, L), jnp.float32),
        pltpu.VMEM_SHARED((NS, L), jnp.float32),
        pltpu.SemaphoreType.DMA,
    ],
)
def _arg_loss_body(yt_hbm, yp_hbm, out_hbm, yt_v, idx_v, val_v, part_v,
                   all_v, shared_v, gsem):
    s = lax.axis_index("s")
    pltpu.sync_copy(yt_hbm.at[pl.ds(s * (PER * L), PER * L)], yt_v)
    lane = lax.iota(jnp.int32, L)
    lane_base = lane * (A * C)
    for k in range(PER):
        a = s * PER + k
        a_base = (
            lax.shift_left(lax.shift_right_logical(a, 3), 15)
            + lax.shift_left(lax.bitwise_and(a, 7), 7)
        )
        yt16 = yt_v[pl.ds(k * L, L)]
        safe = jnp.maximum(yt16, 0)
        col = lax.shift_left(lax.shift_right_logical(safe, 7), 10) + \
            lax.bitwise_and(safe, 127)
        idx_v[pl.ds(k * L, L)] = lane_base + a_base + col
    pltpu.async_copy(yp_hbm.at[idx_v], val_v, gsem).wait()
    acc = jnp.zeros((L,), jnp.float32)
    for k in range(PER):
        g = val_v[pl.ds(k * L, L)]
        m = yt_v[pl.ds(k * L, L)] >= 0
        acc = acc + jnp.where(m, g, jnp.zeros((L,), jnp.float32))
    part_v[...] = acc
    pltpu.sync_copy(part_v, shared_v.at[s])
    plsc.subcore_barrier()

    @pl.when(s == 0)
    def _():
        pltpu.sync_copy(shared_v, all_v)
        out = jnp.zeros((L,), jnp.float32)
        for j in range(NS):
            out = out + all_v[j, :]
        part_v[...] = -out
        pltpu.sync_copy(part_v, out_hbm)


def kernel(y_pred, y_true):
    yt = jnp.squeeze(y_true, axis=1).astype(jnp.int32).T.reshape(-1)
    yp = y_pred.reshape(B * A // 8, 8, C // 128, 128).transpose(0, 2, 1, 3).reshape(-1)
    return _arg_loss_body(yt, yp)
